# baseline (speedup 1.0000x reference)
import functools

import jax
import jax.numpy as jnp
from jax.experimental import pallas as pl
from jax.experimental.pallas import tpu as pltpu

_B, _S, _D = 4, 4096, 128
_Q, _K = 4, 512
_W = 0.25
_N = _B * _S
_TB = 4096
_GRID = _N // _TB
_LOSS_SCALE = _W / (_B * _S * _D)


def _rvq_block(x_ref, cbm2_ref, cbh_ref, cbm_ref, cbl_ref,
               q_ref, idx_ref, loss_ref):
    step = pl.program_id(0)
    x = x_ref[...]
    residual = x
    quantized = jnp.zeros_like(x)
    loss = jnp.zeros((), jnp.float32)
    iota_k = jax.lax.broadcasted_iota(jnp.int32, (_TB, _K), 1)
    idx_cols = []
    r2 = jnp.sum(x * x, axis=-1, keepdims=True)
    for i in range(_Q):
        cbm2 = cbm2_ref[i]
        c2 = jnp.sum(cbm2 * cbm2, axis=-1) * 0.25
        cross2 = jax.lax.dot_general(
            residual, cbm2, (((1,), (1,)), ((), ())),
            preferred_element_type=jnp.float32)
        d2 = r2 + cross2 + c2[None, :]
        idx = jnp.argmin(d2, axis=-1).reshape(_TB, 1)
        idx_cols.append(idx)
        onehot = (iota_k == idx).astype(jnp.bfloat16)
        q_step = jnp.dot(onehot, cbh_ref[i], preferred_element_type=jnp.float32)
        q_step = q_step + jnp.dot(onehot, cbm_ref[i],
                                  preferred_element_type=jnp.float32)
        q_step = q_step + jnp.dot(onehot, cbl_ref[i],
                                  preferred_element_type=jnp.float32)
        residual = residual - q_step
        quantized = quantized + q_step
        r2 = jnp.sum(residual * residual, axis=-1, keepdims=True)
        loss = loss + jnp.sum(r2)
    q_ref[...] = quantized
    idx_ref[...] = jnp.concatenate(idx_cols, axis=1)

    @pl.when(step == 0)
    def _init():
        loss_ref[...] = jnp.zeros((1, 1), jnp.float32)

    loss_ref[...] += (loss * _LOSS_SCALE).reshape(1, 1)


@jax.jit
def kernel(x, codebooks):
    xf = x.reshape(_N, _D)
    cbm2 = -2.0 * codebooks

    def _trunc16(v):
        u = jax.lax.bitcast_convert_type(v, jnp.uint32)
        v_f = jax.lax.bitcast_convert_type(
            u & jnp.uint32(0xFFFF0000), jnp.float32)
        v_bf = jax.lax.bitcast_convert_type(
            (u >> 16).astype(jnp.uint16), jnp.bfloat16)
        return v_f, v_bf

    hi_f, cb_hi = _trunc16(codebooks)
    rem1 = codebooks - hi_f
    mid_f, cb_mid = _trunc16(rem1)
    cb_lo = (rem1 - mid_f).astype(jnp.bfloat16)
    cb_spec = pl.BlockSpec((_Q, _K, _D), lambda i: (0, 0, 0))
    quantized, indices, loss = pl.pallas_call(
        _rvq_block,
        grid=(_GRID,),
        in_specs=[
            pl.BlockSpec((_TB, _D), lambda i: (i, 0)),
            cb_spec, cb_spec, cb_spec, cb_spec,
        ],
        out_specs=[
            pl.BlockSpec((_TB, _D), lambda i: (i, 0)),
            pl.BlockSpec((_TB, _Q), lambda i: (i, 0)),
            pl.BlockSpec((1, 1), lambda i: (0, 0)),
        ],
        out_shape=[
            jax.ShapeDtypeStruct((_N, _D), jnp.float32),
            jax.ShapeDtypeStruct((_N, _Q), jnp.int32),
            jax.ShapeDtypeStruct((1, 1), jnp.float32),
        ],
        compiler_params=pltpu.CompilerParams(
            dimension_semantics=("arbitrary",),
        ),
    )(xf, cbm2, cb_hi, cb_mid, cb_lo)
    return (quantized.reshape(_B, _S, _D),
            indices.reshape(_B, _S, _Q),
            loss.reshape(()))


# speedup vs baseline: 1.0350x; 1.0350x over previous
import functools

import jax
import jax.numpy as jnp
from jax.experimental import pallas as pl
from jax.experimental.pallas import tpu as pltpu

_B, _S, _D = 4, 4096, 128
_Q, _K = 4, 512
_W = 0.25
_N = _B * _S
_TB = 4096
_GRID = _N // _TB
_LOSS_SCALE = _W / (_B * _S * _D)


def _rvq_block(x_ref, cb_ref, q_ref, idx_ref, loss_ref):
    step = pl.program_id(0)
    x = x_ref[...]
    residual = x
    quantized = jnp.zeros_like(x)
    loss = jnp.zeros((), jnp.float32)
    iota_k = jax.lax.broadcasted_iota(jnp.int32, (_TB, _K), 1)
    idx_cols = []
    r2 = jnp.sum(x * x, axis=-1, keepdims=True)
    for i in range(_Q):
        cb = cb_ref[i]
        c2 = jnp.sum(cb * cb, axis=-1)
        cross2 = jax.lax.dot_general(
            residual, -2.0 * cb, (((1,), (1,)), ((), ())),
            preferred_element_type=jnp.float32)
        d2 = r2 + cross2 + c2[None, :]
        idx = jnp.argmin(d2, axis=-1).reshape(_TB, 1)
        idx_cols.append(idx)
        onehot = (iota_k == idx).astype(jnp.bfloat16)
        cb_hi = cb.astype(jnp.bfloat16)
        rem1 = cb - cb_hi.astype(jnp.float32)
        cb_mid = rem1.astype(jnp.bfloat16)
        cb_lo = (rem1 - cb_mid.astype(jnp.float32)).astype(jnp.bfloat16)
        q_step = jnp.dot(onehot, cb_hi, preferred_element_type=jnp.float32)
        q_step = q_step + jnp.dot(onehot, cb_mid,
                                  preferred_element_type=jnp.float32)
        q_step = q_step + jnp.dot(onehot, cb_lo,
                                  preferred_element_type=jnp.float32)
        residual = residual - q_step
        quantized = quantized + q_step
        r2 = jnp.sum(residual * residual, axis=-1, keepdims=True)
        loss = loss + jnp.sum(r2)
    q_ref[...] = quantized
    idx_ref[...] = jnp.concatenate(idx_cols, axis=1)

    @pl.when(step == 0)
    def _init():
        loss_ref[...] = jnp.zeros((1, 1), jnp.float32)

    loss_ref[...] += (loss * _LOSS_SCALE).reshape(1, 1)


@jax.jit
def kernel(x, codebooks):
    xf = x.reshape(_N, _D)
    quantized, indices, loss = pl.pallas_call(
        _rvq_block,
        grid=(_GRID,),
        in_specs=[
            pl.BlockSpec((_TB, _D), lambda i: (i, 0)),
            pl.BlockSpec((_Q, _K, _D), lambda i: (0, 0, 0)),
        ],
        out_specs=[
            pl.BlockSpec((_TB, _D), lambda i: (i, 0)),
            pl.BlockSpec((_TB, _Q), lambda i: (i, 0)),
            pl.BlockSpec((1, 1), lambda i: (0, 0)),
        ],
        out_shape=[
            jax.ShapeDtypeStruct((_N, _D), jnp.float32),
            jax.ShapeDtypeStruct((_N, _Q), jnp.int32),
            jax.ShapeDtypeStruct((1, 1), jnp.float32),
        ],
        compiler_params=pltpu.CompilerParams(
            dimension_semantics=("arbitrary",),
        ),
    )(xf, codebooks)
    return (quantized.reshape(_B, _S, _D),
            indices.reshape(_B, _S, _Q),
            loss.reshape(()))


# speedup vs baseline: 1.9240x; 1.8589x over previous
import functools

import jax
import jax.numpy as jnp
from jax.experimental import pallas as pl
from jax.experimental.pallas import tpu as pltpu

_B, _S, _D = 4, 4096, 128
_Q, _K = 4, 512
_W = 0.25
_N = _B * _S
_TB = 4096
_GRID = _N // _TB
_LOSS_SCALE = _W / (_B * _S * _D)


def _rvq_block(x_ref, cb_ref, q_ref, idx_ref, loss_ref):
    step = pl.program_id(0)
    x = x_ref[...]
    residual = x
    quantized = jnp.zeros_like(x)
    loss = jnp.zeros((), jnp.float32)
    iota_k = jax.lax.broadcasted_iota(jnp.int32, (_TB, _K), 1)
    idx_cols = []
    r2 = jnp.sum(x * x, axis=-1, keepdims=True)
    for i in range(_Q):
        cb = cb_ref[i]
        c2 = jnp.sum(cb * cb, axis=-1)
        cross2 = jax.lax.dot_general(
            residual, -2.0 * cb, (((1,), (1,)), ((), ())),
            preferred_element_type=jnp.float32)
        d2 = r2 + cross2 + c2[None, :]
        idx = jnp.argmin(d2, axis=-1).reshape(_TB, 1)
        idx_cols.append(idx)
        onehot = (iota_k == idx).astype(jnp.bfloat16)
        cb_hi = cb.astype(jnp.bfloat16)
        rem1 = cb - cb_hi.astype(jnp.float32)
        cb_mid = rem1.astype(jnp.bfloat16)
        cb_lo = (rem1 - cb_mid.astype(jnp.float32)).astype(jnp.bfloat16)
        cb3w = jnp.concatenate([cb_hi, cb_mid, cb_lo], axis=1)
        q3 = jnp.dot(onehot, cb3w, preferred_element_type=jnp.float32)
        q_step = (q3[:, 0:_D] + q3[:, _D:2 * _D]) + q3[:, 2 * _D:3 * _D]
        residual = residual - q_step
        quantized = quantized + q_step
        r2 = jnp.sum(residual * residual, axis=-1, keepdims=True)
        loss = loss + jnp.sum(r2)
    q_ref[...] = quantized
    idx_ref[...] = jnp.concatenate(idx_cols, axis=1)

    @pl.when(step == 0)
    def _init():
        loss_ref[...] = jnp.zeros((1, 1), jnp.float32)

    loss_ref[...] += (loss * _LOSS_SCALE).reshape(1, 1)


@jax.jit
def kernel(x, codebooks):
    xf = x.reshape(_N, _D)
    quantized, indices, loss = pl.pallas_call(
        _rvq_block,
        grid=(_GRID,),
        in_specs=[
            pl.BlockSpec((_TB, _D), lambda i: (i, 0)),
            pl.BlockSpec((_Q, _K, _D), lambda i: (0, 0, 0)),
        ],
        out_specs=[
            pl.BlockSpec((_TB, _D), lambda i: (i, 0)),
            pl.BlockSpec((_TB, _Q), lambda i: (i, 0)),
            pl.BlockSpec((1, 1), lambda i: (0, 0)),
        ],
        out_shape=[
            jax.ShapeDtypeStruct((_N, _D), jnp.float32),
            jax.ShapeDtypeStruct((_N, _Q), jnp.int32),
            jax.ShapeDtypeStruct((1, 1), jnp.float32),
        ],
        compiler_params=pltpu.CompilerParams(
            dimension_semantics=("arbitrary",),
        ),
    )(xf, codebooks)
    return (quantized.reshape(_B, _S, _D),
            indices.reshape(_B, _S, _Q),
            loss.reshape(()))
